# Initial kernel scaffold; baseline (speedup 1.0000x reference)
import jax
import jax.numpy as jnp
from jax.experimental import pallas as pl


def kernel(halide_matrices, halide_features, ligand_matrices, ligand_features, base_matrices, base_features, additive_matrices, additive_features, W_in, b_in, W_self, W_msg, b_msg, W1, b1, W2, b2):
    raise NotImplementedError("write your pallas kernel here")


# speedup vs baseline: 1.8461x; 1.8461x over previous
import jax
import jax.numpy as jnp
from jax.experimental import pallas as pl
from jax.experimental.pallas import tpu as pltpu

_B, _N, _F, _MS, _PASSES = 2048, 32, 28, 128, 3
_BT = 128


def _dot(a, b):
    return jax.lax.dot_general(a, b, (((1,), (0,)), ((), ())),
                               preferred_element_type=jnp.float32)


def _bdot(a, b):
    return jax.lax.dot_general(a, b, (((2,), (1,)), ((0,), (0,))),
                               preferred_element_type=jnp.float32)


def _tile_kernel(ah_ref, xh_ref, al_ref, xl_ref, ab_ref, xb_ref, aa_ref, xa_ref,
                 win_ref, bin_ref, wself_ref, wmsg_ref, bmsg_ref,
                 w1_ref, b1_ref, w2_ref, b2_ref, out_ref):
    win = win_ref[...]
    b_in = bin_ref[...]
    wself = wself_ref[...]
    wmsg = wmsg_ref[...]
    bmsg = bmsg_ref[...]

    hidden = jnp.broadcast_to(b1_ref[...], (_BT, 4 * _MS))
    pairs = ((ah_ref, xh_ref), (al_ref, xl_ref), (ab_ref, xb_ref), (aa_ref, xa_ref))
    for t, (a_ref, x_ref) in enumerate(pairs):
        A = a_ref[...]
        X = x_ref[...].reshape(_BT * _N, _F)
        h = jnp.tanh(_dot(X, win) + b_in)
        for _ in range(_PASSES):
            m = _bdot(A, h.reshape(_BT, _N, _MS)).reshape(_BT * _N, _MS)
            h = jnp.tanh(_dot(h, wself) + _dot(m, wmsg) + bmsg)
        emb = jnp.sum(h.reshape(_BT, _N, _MS), axis=1)
        hidden = hidden + _dot(emb, w1_ref[t * _MS:(t + 1) * _MS, :])
    hidden = jax.nn.relu(hidden)
    y = _dot(hidden, w2_ref[...]) + b2_ref[...]
    out_ref[...] = jnp.abs(y)


def kernel(halide_matrices, halide_features, ligand_matrices, ligand_features,
           base_matrices, base_features, additive_matrices, additive_features,
           W_in, b_in, W_self, W_msg, b_msg, W1, b1, W2, b2):
    grid = (_B // _BT,)
    a_spec = pl.BlockSpec((_BT, _N, _N), lambda i: (i, 0, 0))
    x_spec = pl.BlockSpec((_BT, _N, _F), lambda i: (i, 0, 0))

    def w_spec(shape):
        return pl.BlockSpec(shape, lambda i: tuple(0 for _ in shape))

    out = pl.pallas_call(
        _tile_kernel,
        grid=grid,
        in_specs=[a_spec, x_spec, a_spec, x_spec, a_spec, x_spec, a_spec, x_spec,
                  w_spec((_F, _MS)), w_spec((1, _MS)),
                  w_spec((_MS, _MS)), w_spec((_MS, _MS)), w_spec((1, _MS)),
                  w_spec((4 * _MS, 4 * _MS)), w_spec((1, 4 * _MS)),
                  w_spec((4 * _MS, 1)), w_spec((1, 1))],
        out_specs=pl.BlockSpec((_BT, 1), lambda i: (i, 0)),
        out_shape=jax.ShapeDtypeStruct((_B, 1), jnp.float32),
        compiler_params=pltpu.CompilerParams(
            dimension_semantics=("arbitrary",)),
    )(halide_matrices, halide_features, ligand_matrices, ligand_features,
      base_matrices, base_features, additive_matrices, additive_features,
      W_in, b_in.reshape(1, _MS), W_self, W_msg, b_msg.reshape(1, _MS),
      W1, b1.reshape(1, 4 * _MS), W2, b2.reshape(1, 1))
    return out.reshape(-1)


# speedup vs baseline: 1.9173x; 1.0386x over previous
import jax
import jax.numpy as jnp
from jax.experimental import pallas as pl
from jax.experimental.pallas import tpu as pltpu

_B, _N, _F, _MS, _PASSES = 2048, 32, 28, 128, 3
_NT = 4
_NA = _NT * _N
_BT = 128


def _dot(a, b):
    return jax.lax.dot_general(a, b, (((1,), (0,)), ((), ())),
                               preferred_element_type=jnp.float32)


def _bdot(a, b):
    return jax.lax.dot_general(a, b, (((2,), (1,)), ((0,), (0,))),
                               preferred_element_type=jnp.float32)


def _tile_kernel(d_ref, xt_ref, win_ref, bin_ref, wself_ref, wmsg_ref,
                 bmsg_ref, w1_ref, b1_ref, w2_ref, b2_ref, out_ref):
    D = d_ref[...]
    h = jnp.tanh(jax.lax.dot_general(
        xt_ref[...], win_ref[...], (((1,), (0,)), ((), ())),
        preferred_element_type=jnp.float32) + bin_ref[...])
    wself = wself_ref[...]
    wmsg = wmsg_ref[...]
    bmsg = bmsg_ref[...]
    h = h.reshape(_BT * _NA, _MS)
    for _ in range(_PASSES):
        m = _bdot(D, h.reshape(_BT, _NA, _MS)).reshape(_BT * _NA, _MS)
        h = jnp.tanh(_dot(h, wself) + _dot(m, wmsg) + bmsg)
    embs = jnp.sum(h.reshape(_BT, _NT, _N, _MS), axis=2)
    hidden = jnp.broadcast_to(b1_ref[...], (_BT, _NT * _MS))
    for t in range(_NT):
        hidden = hidden + _dot(embs[:, t, :], w1_ref[t * _MS:(t + 1) * _MS, :])
    hidden = jax.nn.relu(hidden)
    y = _dot(hidden, w2_ref[...]) + b2_ref[...]
    out_ref[...] = jnp.abs(y)


def kernel(halide_matrices, halide_features, ligand_matrices, ligand_features,
           base_matrices, base_features, additive_matrices, additive_features,
           W_in, b_in, W_self, W_msg, b_msg, W1, b1, W2, b2):
    mats = (halide_matrices, ligand_matrices, base_matrices, additive_matrices)
    feats = (halide_features, ligand_features, base_features, additive_features)
    D = jnp.concatenate(
        [jnp.pad(a, ((0, 0), (0, 0), (t * _N, _NA - (t + 1) * _N)))
         for t, a in enumerate(mats)], axis=1)
    X_T = jnp.concatenate([x.transpose(0, 2, 1) for x in feats], axis=2)

    grid = (_B // _BT,)

    def w_spec(shape):
        return pl.BlockSpec(shape, lambda i: tuple(0 for _ in shape))

    out = pl.pallas_call(
        _tile_kernel,
        grid=grid,
        in_specs=[pl.BlockSpec((_BT, _NA, _NA), lambda i: (i, 0, 0)),
                  pl.BlockSpec((_BT, _F, _NA), lambda i: (i, 0, 0)),
                  w_spec((_F, _MS)), w_spec((1, _MS)),
                  w_spec((_MS, _MS)), w_spec((_MS, _MS)), w_spec((1, _MS)),
                  w_spec((_NT * _MS, _NT * _MS)), w_spec((1, _NT * _MS)),
                  w_spec((_NT * _MS, 1)), w_spec((1, 1))],
        out_specs=pl.BlockSpec((_BT, 1), lambda i: (i, 0)),
        out_shape=jax.ShapeDtypeStruct((_B, 1), jnp.float32),
        compiler_params=pltpu.CompilerParams(
            dimension_semantics=("arbitrary",)),
    )(D, X_T,
      W_in, b_in.reshape(1, _MS), W_self, W_msg, b_msg.reshape(1, _MS),
      W1, b1.reshape(1, _NT * _MS), W2, b2.reshape(1, 1))
    return out.reshape(-1)


# speedup vs baseline: 1.9615x; 1.0231x over previous
import jax
import jax.numpy as jnp
from jax.experimental import pallas as pl
from jax.experimental.pallas import tpu as pltpu

_B, _N, _F, _MS, _PASSES = 2048, 32, 28, 128, 3
_NT = 4
_NA = _NT * _N
_BT = 128


def _dot(a, b):
    return jax.lax.dot_general(a, b, (((1,), (0,)), ((), ())),
                               preferred_element_type=jnp.float32)


def _bdot(a, b):
    return jax.lax.dot_general(a, b, (((2,), (1,)), ((0,), (0,))),
                               preferred_element_type=jnp.float32)


def _tile_kernel(a_ref, x_ref, wstk_ref, bin4_ref, wself_ref, wmsg_ref,
                 bmsg_ref, w1_ref, b1_ref, w2_ref, b2_ref, out_ref, d_ref):
    @pl.when(pl.program_id(0) == 0)
    def _():
        d_ref[...] = jnp.zeros_like(d_ref)

    for t in range(_NT):
        d_ref[:, t * _N:(t + 1) * _N, t * _N:(t + 1) * _N] = \
            a_ref[:, :, t * _N:(t + 1) * _N]

    x2 = x_ref[...].reshape(_BT * _N, _NT * _F)
    h0 = jnp.tanh(_dot(x2, wstk_ref[...]) + bin4_ref[...])
    h = jnp.concatenate(
        [h0[:, t * _MS:(t + 1) * _MS].reshape(_BT, _N, _MS)
         for t in range(_NT)], axis=1)
    h = h.reshape(_BT * _NA, _MS)

    D = d_ref[...]
    wself = wself_ref[...]
    wmsg = wmsg_ref[...]
    bmsg = bmsg_ref[...]
    for _ in range(_PASSES):
        m = _bdot(D, h.reshape(_BT, _NA, _MS)).reshape(_BT * _NA, _MS)
        h = jnp.tanh(_dot(h, wself) + _dot(m, wmsg) + bmsg)

    embs = jnp.sum(h.reshape(_BT, _NT, _N, _MS), axis=2)
    hidden = jnp.broadcast_to(b1_ref[...], (_BT, _NT * _MS))
    for t in range(_NT):
        hidden = hidden + _dot(embs[:, t, :], w1_ref[t * _MS:(t + 1) * _MS, :])
    hidden = jax.nn.relu(hidden)
    y = _dot(hidden, w2_ref[...]) + b2_ref[...]
    out_ref[...] = jnp.abs(y)


def kernel(halide_matrices, halide_features, ligand_matrices, ligand_features,
           base_matrices, base_features, additive_matrices, additive_features,
           W_in, b_in, W_self, W_msg, b_msg, W1, b1, W2, b2):
    mats = (halide_matrices, ligand_matrices, base_matrices, additive_matrices)
    feats = (halide_features, ligand_features, base_features, additive_features)
    A_cat = jnp.concatenate(mats, axis=2)
    X_cat = jnp.concatenate(feats, axis=2)
    W_stack = jnp.zeros((_NT * _F, _NT * _MS), jnp.float32)
    for t in range(_NT):
        W_stack = W_stack.at[t * _F:(t + 1) * _F,
                             t * _MS:(t + 1) * _MS].set(W_in)
    b_in4 = jnp.tile(b_in, _NT).reshape(1, _NT * _MS)

    grid = (_B // _BT,)

    def w_spec(shape):
        return pl.BlockSpec(shape, lambda i: tuple(0 for _ in shape))

    out = pl.pallas_call(
        _tile_kernel,
        grid=grid,
        in_specs=[pl.BlockSpec((_BT, _N, _NT * _N), lambda i: (i, 0, 0)),
                  pl.BlockSpec((_BT, _N, _NT * _F), lambda i: (i, 0, 0)),
                  w_spec((_NT * _F, _NT * _MS)), w_spec((1, _NT * _MS)),
                  w_spec((_MS, _MS)), w_spec((_MS, _MS)), w_spec((1, _MS)),
                  w_spec((_NT * _MS, _NT * _MS)), w_spec((1, _NT * _MS)),
                  w_spec((_NT * _MS, 1)), w_spec((1, 1))],
        out_specs=pl.BlockSpec((_BT, 1), lambda i: (i, 0)),
        out_shape=jax.ShapeDtypeStruct((_B, 1), jnp.float32),
        scratch_shapes=[pltpu.VMEM((_BT, _NA, _NA), jnp.float32)],
        compiler_params=pltpu.CompilerParams(
            dimension_semantics=("arbitrary",)),
    )(A_cat, X_cat,
      W_stack, b_in4, W_self, W_msg, b_msg.reshape(1, _MS),
      W1, b1.reshape(1, _NT * _MS), W2, b2.reshape(1, 1))
    return out.reshape(-1)
